# baseline (speedup 1.0000x reference)
import dataclasses
import functools

import jax
import jax.numpy as jnp
from jax import lax
from jax.experimental import pallas as pl
from jax.experimental.pallas import tpu as pltpu
from jax.experimental.pallas import tpu_sc as plsc

B = 16384
C = 1000
NT = 16
PER_TILE = B // NT
CHUNKS = PER_TILE // 128

_LN2 = 0.6931471805599453
_SQRT2 = 1.4142135


def _log16(x):
    bits = lax.bitcast_convert_type(x, jnp.int32)
    e = ((bits >> 23) & 0xFF) - 127
    m = lax.bitcast_convert_type((bits & 0x007FFFFF) | 0x3F800000, jnp.float32)
    big = m > _SQRT2
    m = jnp.where(big, m * 0.5, m)
    ef = (e + jnp.where(big, 1, 0)).astype(jnp.float32)
    s = (m - 1.0) / (m + 1.0)
    z = s * s
    p = 1.0 + z * (1 / 3 + z * (1 / 5 + z * (1 / 7 + z * (1 / 9))))
    return ef * _LN2 + 2.0 * s * p


def _sc_body(ypf_hbm, yt_hbm, out_hbm, yt_v, idx_v, vals_v, stage_v, sem):
    sid = lax.axis_index("s")
    base = sid * PER_TILE

    pltpu.sync_copy(yt_hbm.at[pl.ds(base, PER_TILE)], yt_v)

    iota = lax.iota(jnp.int32, 16)
    for v in range(PER_TILE // 16):
        rows = base + v * 16 + iota
        flat = rows * C + yt_v[pl.ds(v * 16, 16)]
        idx_v[v // 8, pl.ds((v % 8) * 16, 16)] = flat

    copies = [
        pltpu.async_copy(ypf_hbm.at[idx_v.at[j]], vals_v.at[j], sem)
        for j in range(CHUNKS)
    ]
    for c in copies:
        c.wait()

    acc = jnp.zeros((16,), jnp.float32)
    for j in range(CHUNKS):
        for h in range(8):
            x = vals_v[j, pl.ds(h * 16, 16)] + 1e-7
            acc = acc + _log16(x)

    stage_v[...] = acc
    pltpu.sync_copy(stage_v, out_hbm.at[sid])


def _tc_finish_body(part_ref, out_ref):
    out_ref[0, 0] = jnp.sum(part_ref[...]) * (-1.0 / B)


@jax.jit
def kernel(y_pred, y_true):
    ypf = y_pred.reshape(-1)
    yt = y_true.astype(jnp.int32)
    mesh = plsc.VectorSubcoreMesh(
        core_axis_name="c", subcore_axis_name="s", num_cores=1)
    cp = pltpu.CompilerParams()
    if "needs_layout_passes" in pltpu.CompilerParams.__dataclass_fields__:
        cp = dataclasses.replace(cp, needs_layout_passes=False)
    run = pl.kernel(
        _sc_body,
        out_type=jax.ShapeDtypeStruct((NT, 16), jnp.float32),
        mesh=mesh,
        scratch_types=[
            pltpu.VMEM((PER_TILE,), jnp.int32),
            pltpu.VMEM((CHUNKS, 128), jnp.int32),
            pltpu.VMEM((CHUNKS, 128), jnp.float32),
            pltpu.VMEM((16,), jnp.float32),
            pltpu.SemaphoreType.DMA,
        ],
        compiler_params=cp,
    )
    part = run(ypf, yt)
    loss = pl.pallas_call(
        _tc_finish_body,
        out_shape=jax.ShapeDtypeStruct((1, 1), jnp.float32),
        out_specs=pl.BlockSpec(memory_space=pltpu.SMEM),
    )(part)
    return loss[0, 0]


# speedup vs baseline: 1.5860x; 1.5860x over previous
import jax
import jax.numpy as jnp
from jax import lax
from jax.experimental import pallas as pl
from jax.experimental.pallas import tpu as pltpu

B = 16384
C = 1000
BR = 512
NB = B // BR


def _tc_body(yt_ref, yp_ref, out_ref):
    i = pl.program_id(0)
    yt = yt_ref[0, 0, :]
    cols = lax.broadcasted_iota(jnp.int32, (BR, C), 1)
    mask = cols == yt[:, None]
    vals = jnp.sum(jnp.where(mask, yp_ref[...], 0.0), axis=1)
    s = jnp.sum(jnp.log(vals + 1e-7))
    prev = jnp.where(i == 0, 0.0, out_ref[0, 0])
    total = prev + s
    out_ref[0, 0] = jnp.where(i == NB - 1, total * (-1.0 / B), total)


@jax.jit
def kernel(y_pred, y_true):
    yt = y_true.astype(jnp.int32).reshape(NB, 1, BR)
    loss = pl.pallas_call(
        _tc_body,
        grid=(NB,),
        in_specs=[
            pl.BlockSpec((1, 1, BR), lambda i: (i, 0, 0)),
            pl.BlockSpec((BR, C), lambda i: (i, 0)),
        ],
        out_specs=pl.BlockSpec(memory_space=pltpu.SMEM),
        out_shape=jax.ShapeDtypeStruct((1, 1), jnp.float32),
    )(yt, y_pred)
    return loss[0, 0]


# speedup vs baseline: 1.8800x; 1.1854x over previous
import jax
import jax.numpy as jnp
from jax import lax
from jax.experimental import pallas as pl
from jax.experimental.pallas import tpu as pltpu

B = 16384
C = 1000
BR = 2048
NB = B // BR


def _tc_body(yt_ref, yp_ref, out_ref):
    i = pl.program_id(0)
    yt = yt_ref[0, 0, :]
    cols = lax.broadcasted_iota(jnp.int32, (BR, C), 1)
    mask = cols == yt[:, None]
    vals = jnp.sum(jnp.where(mask, yp_ref[...], 0.0), axis=1)
    s = jnp.sum(jnp.log(vals + 1e-7))
    prev = jnp.where(i == 0, 0.0, out_ref[0, 0])
    total = prev + s
    out_ref[0, 0] = jnp.where(i == NB - 1, total * (-1.0 / B), total)


@jax.jit
def kernel(y_pred, y_true):
    yt = y_true.astype(jnp.int32).reshape(NB, 1, BR)
    loss = pl.pallas_call(
        _tc_body,
        grid=(NB,),
        in_specs=[
            pl.BlockSpec((1, 1, BR), lambda i: (i, 0, 0)),
            pl.BlockSpec((BR, C), lambda i: (i, 0)),
        ],
        out_specs=pl.BlockSpec(memory_space=pltpu.SMEM),
        out_shape=jax.ShapeDtypeStruct((1, 1), jnp.float32),
    )(yt, y_pred)
    return loss[0, 0]
